# baseline (speedup 1.0000x reference)
import jax
import jax.numpy as jnp
from jax.experimental import pallas as pl
from jax.experimental.pallas import tpu as pltpu


def _xw_kernel(x_ref, w_ref, h_ref):
    h_ref[...] = jnp.dot(x_ref[...], w_ref[...],
                         preferred_element_type=jnp.float32).astype(jnp.bfloat16)


def _adj_kernel(adj_ref, h_ref, b_ref, out_ref):
    a = adj_ref[...].astype(jnp.bfloat16)
    out_ref[...] = jnp.dot(a, h_ref[...],
                           preferred_element_type=jnp.float32) + b_ref[...]


def kernel(x, adj, W, b):
    n, f = x.shape
    h_dim = W.shape[1]

    bm1 = 1000 if n % 1000 == 0 else n
    h = pl.pallas_call(
        _xw_kernel,
        grid=(n // bm1,),
        in_specs=[
            pl.BlockSpec((bm1, f), lambda i: (i, 0)),
            pl.BlockSpec((f, h_dim), lambda i: (0, 0)),
        ],
        out_specs=pl.BlockSpec((bm1, h_dim), lambda i: (i, 0)),
        out_shape=jax.ShapeDtypeStruct((n, h_dim), jnp.bfloat16),
    )(x, W)

    bm2 = 400 if n % 400 == 0 else n
    out = pl.pallas_call(
        _adj_kernel,
        grid=(n // bm2,),
        in_specs=[
            pl.BlockSpec((bm2, n), lambda i: (i, 0)),
            pl.BlockSpec((n, h_dim), lambda i: (0, 0)),
            pl.BlockSpec((1, h_dim), lambda i: (0, 0)),
        ],
        out_specs=pl.BlockSpec((bm2, h_dim), lambda i: (i, 0)),
        out_shape=jax.ShapeDtypeStruct((n, h_dim), jnp.float32),
        compiler_params=pltpu.CompilerParams(
            vmem_limit_bytes=120 * 1024 * 1024,
        ),
    )(adj, h, b.reshape(1, h_dim))
    return out


# speedup vs baseline: 1.0615x; 1.0615x over previous
import jax
import jax.numpy as jnp
from jax.experimental import pallas as pl
from jax.experimental.pallas import tpu as pltpu


def _make_kernel(n_hsteps, chunk):
    def _fused_kernel(x_ref, w_ref, adj_ref, b_ref, out_ref, h_ref):
        i = pl.program_id(0)

        @pl.when(i < n_hsteps)
        def _():
            h_ref[pl.ds(i * chunk, chunk), :] = jnp.dot(
                x_ref[...], w_ref[...],
                preferred_element_type=jnp.float32).astype(jnp.bfloat16)

        @pl.when(i >= n_hsteps)
        def _():
            a = adj_ref[...].astype(jnp.bfloat16)
            out_ref[...] = jnp.dot(
                a, h_ref[...],
                preferred_element_type=jnp.float32) + b_ref[...]

    return _fused_kernel


def kernel(x, adj, W, b):
    n, f = x.shape
    h_dim = W.shape[1]

    n_hsteps = 5 if n % (5 * 8) == 0 else 1
    chunk = n // n_hsteps
    bm = 400 if n % 400 == 0 else n
    n_msteps = n // bm
    grid = (n_hsteps + n_msteps,)

    out = pl.pallas_call(
        _make_kernel(n_hsteps, chunk),
        grid=grid,
        in_specs=[
            pl.BlockSpec((chunk, f), lambda i: (jnp.minimum(i, n_hsteps - 1), 0)),
            pl.BlockSpec((f, h_dim), lambda i: (0, 0)),
            pl.BlockSpec((bm, n), lambda i: (jnp.maximum(i - n_hsteps, 0), 0)),
            pl.BlockSpec((1, h_dim), lambda i: (0, 0)),
        ],
        out_specs=pl.BlockSpec((bm, h_dim), lambda i: (jnp.maximum(i - n_hsteps, 0), 0)),
        out_shape=jax.ShapeDtypeStruct((n, h_dim), jnp.float32),
        scratch_shapes=[pltpu.VMEM((n, h_dim), jnp.bfloat16)],
        compiler_params=pltpu.CompilerParams(
            dimension_semantics=("arbitrary",),
        ),
    )(x, W, adj, b.reshape(1, h_dim))
    return out
